# Initial kernel scaffold; baseline (speedup 1.0000x reference)
import jax
import jax.numpy as jnp
from jax.experimental import pallas as pl


def kernel(distances, hierarchy, neighbour_indices, truth_indices):
    raise NotImplementedError("write your pallas kernel here")


# speedup vs baseline: 228.9986x; 228.9986x over previous
import functools

import jax
import jax.numpy as jnp
from jax import lax
from jax.experimental import pallas as pl
from jax.experimental.pallas import tpu as pltpu
from jax.experimental.pallas import tpu_sc as plsc

N = 100000
K = 64
NC, NS, L = 2, 16, 16
NW = NC * NS
R = 80
NB = N // R
assert N % R == 0 and R % L == 0


def _sc_body(d_hbm, n_hbm, h_hbm, t_hbm, out_hbm, table_v, d_blk, n_blk,
             h_blk, out_stage):
    wid = lax.axis_index("s") * NC + lax.axis_index("c")
    pltpu.sync_copy(t_hbm, table_v)

    zero = jnp.zeros((L,), jnp.float32)
    nb = (NB - 1 - wid) // NW + 1

    def blk_body(j, carry):
        acc_pot, acc_pen = carry
        b = wid + j * NW
        pltpu.sync_copy(d_hbm.at[pl.ds(b * (R * K), R * K)], d_blk)
        pltpu.sync_copy(n_hbm.at[pl.ds(b * (R * K), R * K)], n_blk)
        pltpu.sync_copy(h_hbm.at[pl.ds(b * R, R)], h_blk)

        def h_body(jv, acc):
            x = h_blk[pl.ds(jv * L, L)]
            hw = (1.0 / (1.0 + jnp.exp(-x)) + 1.0) * 0.5
            h_blk[pl.ds(jv * L, L)] = hw
            return acc + hw
        acc_pen = lax.fori_loop(0, R // L, h_body, acc_pen)

        def row_body(r, acc):
            grow = b * R + r
            ftv = plsc.load_gather(table_v, [jnp.full((L,), grow, jnp.int32)])
            hv = plsc.load_gather(h_blk, [jnp.full((L,), r, jnp.int32)])
            accrow = zero
            for k4 in range(K // L):
                off = r * K + k4 * L
                idx = n_blk[pl.ds(off, L)]
                gt = plsc.load_gather(table_v, [idx])
                dv = d_blk[pl.ds(off, L)]
                e = jnp.exp(dv * -3.0)
                accrow = accrow + jnp.where(ftv == gt, 0.5 - 0.5 * e, 0.5 * e)
            return acc + hv * hv * accrow
        acc_pot = lax.fori_loop(0, R, row_body, acc_pot)
        return acc_pot, acc_pen

    acc_pot, acc_pen = lax.fori_loop(0, nb, blk_body, (zero, zero))
    out_stage[0, :] = acc_pot
    out_stage[1, :] = acc_pen
    pltpu.sync_copy(out_stage, out_hbm.at[wid])


@functools.partial(jax.jit, static_argnames=())
def _sc_loss_partials(d_flat, n_flat, h_flat, t_flat):
    mesh = plsc.VectorSubcoreMesh(core_axis_name="c", subcore_axis_name="s")
    return pl.kernel(
        _sc_body,
        out_type=jax.ShapeDtypeStruct((NW, 2, L), jnp.float32),
        mesh=mesh,
        compiler_params=pltpu.CompilerParams(needs_layout_passes=False),
        scratch_types=[
            pltpu.VMEM((N,), jnp.int32),
            pltpu.VMEM((R * K,), jnp.float32),
            pltpu.VMEM((R * K,), jnp.int32),
            pltpu.VMEM((R,), jnp.float32),
            pltpu.VMEM((2, L), jnp.float32),
        ],
    )(d_flat, n_flat, h_flat, t_flat)


def kernel(distances, hierarchy, neighbour_indices, truth_indices):
    assert distances.shape == (N, K)
    d_flat = distances.reshape(N * K)
    n_flat = neighbour_indices.reshape(N * K)
    h_flat = hierarchy.reshape(N)
    t_flat = truth_indices.reshape(N)
    parts = _sc_loss_partials(d_flat, n_flat, h_flat, t_flat)
    pot_sum = jnp.sum(parts[:, 0, :])
    h_sum = jnp.sum(parts[:, 1, :])
    lossval = (1.0 - h_sum / N) + pot_sum / (N * K)
    return (distances, lossval)


# speedup vs baseline: 363.1413x; 1.5858x over previous
import jax
import jax.numpy as jnp
from jax import lax
from jax.experimental import pallas as pl
from jax.experimental.pallas import tpu as pltpu
from jax.experimental.pallas import tpu_sc as plsc

N = 100000
K = 64
NC, NS, L = 2, 16, 16
NW = NC * NS
C = 128
KH = K // 2
NBLK = N // C
TAIL0 = NBLK * C
TAIL = N - TAIL0


def _sigmoid_h(x):
    return (1.0 / (1.0 + jnp.exp(-x)) + 1.0) * 0.5


def _sc_body(dT_hbm, nT_hbm, h_hbm, t_hbm, dtl_hbm, ntl_hbm, out_hbm,
             table_v, d_blk, n_blk, h_blk, htl_v, dtl_v, ntl_v, out_stage):
    wid = lax.axis_index("s") * NC + lax.axis_index("c")
    pltpu.sync_copy(t_hbm, table_v)

    zero = jnp.zeros((L,), jnp.float32)
    nb = (NBLK - 1 - wid) // NW + 1

    def half_step(d_half, n_half, c0, acc_pot):
        def g_body(g, acc):
            ftv = table_v[pl.ds(c0 + g * L, L)]
            hw = h_blk[pl.ds(g * L, L)]
            acc16 = zero
            for k in range(KH):
                idx = n_half[k, pl.ds(g * L, L)]
                gt = plsc.load_gather(table_v, [idx])
                dv = d_half[k, pl.ds(g * L, L)]
                e = jnp.exp(dv * -3.0)
                acc16 = acc16 + jnp.where(ftv == gt, 0.5 - 0.5 * e, 0.5 * e)
            return acc + (hw * hw) * acc16
        return lax.fori_loop(0, C // L, g_body, acc_pot)

    def blk_body(j, carry):
        acc_pot, acc_pen = carry
        blk = wid + j * NW
        c0 = blk * C
        pltpu.sync_copy(h_hbm.at[pl.ds(c0, C)], h_blk)

        def h_body(g, acc):
            hw = _sigmoid_h(h_blk[pl.ds(g * L, L)])
            h_blk[pl.ds(g * L, L)] = hw
            return acc + hw
        acc_pen = lax.fori_loop(0, C // L, h_body, acc_pen)

        for half in range(2):
            pltpu.sync_copy(dT_hbm.at[pl.ds(half * KH, KH), pl.ds(c0, C)],
                            d_blk.at[half])
            pltpu.sync_copy(nT_hbm.at[pl.ds(half * KH, KH), pl.ds(c0, C)],
                            n_blk.at[half])
            acc_pot = half_step(d_blk.at[half], n_blk.at[half], c0, acc_pot)
        return acc_pot, acc_pen

    acc_pot, acc_pen = lax.fori_loop(0, nb, blk_body, (zero, zero))
    out_stage[0, :] = acc_pot
    out_stage[1, :] = acc_pen

    @pl.when(wid == NW - 1)
    def _tail():
        pltpu.sync_copy(dtl_hbm, dtl_v)
        pltpu.sync_copy(ntl_hbm, ntl_v)
        pltpu.sync_copy(h_hbm.at[pl.ds(TAIL0, TAIL)], htl_v)

        def row_body(r, carry):
            acc_p, acc_h = carry
            ftv = plsc.load_gather(
                table_v, [jnp.full((L,), TAIL0 + r, jnp.int32)])
            hv = _sigmoid_h(plsc.load_gather(htl_v,
                                             [jnp.full((L,), r, jnp.int32)]))
            accrow = zero
            for g in range(K // L):
                idx = ntl_v[pl.ds(r * K + g * L, L)]
                gt = plsc.load_gather(table_v, [idx])
                dv = dtl_v[pl.ds(r * K + g * L, L)]
                e = jnp.exp(dv * -3.0)
                accrow = accrow + jnp.where(ftv == gt, 0.5 - 0.5 * e, 0.5 * e)
            return acc_p + hv * hv * accrow, acc_h + hv * (1.0 / L)

        acc_p, acc_h = lax.fori_loop(0, TAIL, row_body, (zero, zero))
        out_stage[0, :] = out_stage[0, :] + acc_p
        out_stage[1, :] = out_stage[1, :] + acc_h

    pltpu.sync_copy(out_stage, out_hbm.at[wid])


@jax.jit
def _sc_loss_partials(dT, nT, h_flat, t_flat, d_tail, n_tail):
    mesh = plsc.VectorSubcoreMesh(core_axis_name="c", subcore_axis_name="s")
    return pl.kernel(
        _sc_body,
        out_type=jax.ShapeDtypeStruct((NW, 2, L), jnp.float32),
        mesh=mesh,
        compiler_params=pltpu.CompilerParams(needs_layout_passes=False),
        scratch_types=[
            pltpu.VMEM((N,), jnp.int32),
            pltpu.VMEM((2, KH, C), jnp.float32),
            pltpu.VMEM((2, KH, C), jnp.int32),
            pltpu.VMEM((C,), jnp.float32),
            pltpu.VMEM((TAIL,), jnp.float32),
            pltpu.VMEM((TAIL * K,), jnp.float32),
            pltpu.VMEM((TAIL * K,), jnp.int32),
            pltpu.VMEM((2, L), jnp.float32),
        ],
    )(dT, nT, h_flat, t_flat, d_tail, n_tail)


def kernel(distances, hierarchy, neighbour_indices, truth_indices):
    assert distances.shape == (N, K)
    dT = distances.T
    nT = neighbour_indices.T
    h_flat = hierarchy.reshape(N)
    t_flat = truth_indices.reshape(N)
    d_tail = lax.slice(distances, (TAIL0, 0), (N, K)).reshape(TAIL * K)
    n_tail = lax.slice(neighbour_indices, (TAIL0, 0), (N, K)).reshape(TAIL * K)
    parts = _sc_loss_partials(dT, nT, h_flat, t_flat, d_tail, n_tail)
    pot_sum = jnp.sum(parts[:, 0, :])
    h_sum = jnp.sum(parts[:, 1, :])
    lossval = (1.0 - h_sum / N) + pot_sum / (N * K)
    return (distances, lossval)


# speedup vs baseline: 646.0598x; 1.7791x over previous
import jax
import jax.numpy as jnp
from jax import lax
from jax.experimental import pallas as pl
from jax.experimental.pallas import tpu as pltpu
from jax.experimental.pallas import tpu_sc as plsc

N = 100000
K = 64
NC, NS, L = 2, 16, 16
NW = NC * NS
C = 128
KH = K // 2
NBLK = N // C
TAIL0 = NBLK * C
TAIL = N - TAIL0
QMAX = (NBLK - 1) // NW + 1


def _sigmoid_h(x):
    return (1.0 / (1.0 + jnp.exp(-x)) + 1.0) * 0.5


def _sc_body(dT_hbm, nT_hbm, h_hbm, t_hbm, dtl_hbm, ntl_hbm, out_hbm,
             table_v, d_buf, n_buf, h_buf, htl_v, dtl_v, ntl_v, out_stage,
             sem0, sem1):
    wid = lax.axis_index("s") * NC + lax.axis_index("c")
    pltpu.sync_copy(t_hbm, table_v)

    zero = jnp.zeros((L,), jnp.float32)
    nb = (NBLK - 1 - wid) // NW + 1
    sems = (sem0, sem1)

    def copies(q, half, buf):
        c0 = (wid + q * NW) * C
        return (
            (dT_hbm.at[pl.ds(half * KH, KH), pl.ds(c0, C)], d_buf.at[buf],
             sems[buf]),
            (nT_hbm.at[pl.ds(half * KH, KH), pl.ds(c0, C)], n_buf.at[buf],
             sems[buf]),
            (h_hbm.at[pl.ds(c0, C)], h_buf.at[buf], sems[buf]),
        )

    def start(q, half, buf):
        for src, dst, sem in copies(q, half, buf):
            pltpu.async_copy(src, dst, sem)

    def wait(q, half, buf):
        for src, dst, sem in copies(q, half, buf):
            pltpu.make_async_copy(src, dst, sem).wait()

    def compute(q, buf, carry):
        c0 = (wid + q * NW) * C

        def g_body(g, carry):
            acc_pot, acc_pen = carry
            ftv = table_v[pl.ds(c0 + g * L, L)]
            hw = _sigmoid_h(h_buf[buf, pl.ds(g * L, L)])
            acc16 = zero
            for k in range(KH):
                idx = n_buf[buf, k, pl.ds(g * L, L)]
                gt = plsc.load_gather(table_v, [idx])
                dv = d_buf[buf, k, pl.ds(g * L, L)]
                e = jnp.exp(dv * -3.0)
                acc16 = acc16 + jnp.where(ftv == gt, 0.5 - 0.5 * e, 0.5 * e)
            return acc_pot + (hw * hw) * acc16, acc_pen + 0.5 * hw
        return lax.fori_loop(0, C // L, g_body, carry)

    start(0, 0, 0)

    def q_body(q, carry):
        @pl.when(q < nb)
        def _p0():
            start(q, 1, 1)
            wait(q, 0, 0)
        carry0 = lax.cond(q < nb, lambda c: compute(q, 0, c),
                          lambda c: c, carry)
        @pl.when(q + 1 < nb)
        def _p1a():
            start(q + 1, 0, 0)

        @pl.when(q < nb)
        def _p1b():
            wait(q, 1, 1)
        return lax.cond(q < nb, lambda c: compute(q, 1, c),
                        lambda c: c, carry0)

    acc_pot, acc_pen = lax.fori_loop(0, QMAX, q_body, (zero, zero))
    out_stage[0, :] = acc_pot
    out_stage[1, :] = acc_pen

    @pl.when(wid == NW - 1)
    def _tail():
        pltpu.sync_copy(dtl_hbm, dtl_v)
        pltpu.sync_copy(ntl_hbm, ntl_v)
        pltpu.sync_copy(h_hbm.at[pl.ds(TAIL0, TAIL)], htl_v)

        def row_body(r, carry):
            acc_p, acc_h = carry
            ftv = plsc.load_gather(
                table_v, [jnp.full((L,), TAIL0 + r, jnp.int32)])
            hv = _sigmoid_h(plsc.load_gather(htl_v,
                                             [jnp.full((L,), r, jnp.int32)]))
            accrow = zero
            for g in range(K // L):
                idx = ntl_v[pl.ds(r * K + g * L, L)]
                gt = plsc.load_gather(table_v, [idx])
                dv = dtl_v[pl.ds(r * K + g * L, L)]
                e = jnp.exp(dv * -3.0)
                accrow = accrow + jnp.where(ftv == gt, 0.5 - 0.5 * e, 0.5 * e)
            return acc_p + hv * hv * accrow, acc_h + hv * (1.0 / L)

        acc_p, acc_h = lax.fori_loop(0, TAIL, row_body, (zero, zero))
        out_stage[0, :] = out_stage[0, :] + acc_p
        out_stage[1, :] = out_stage[1, :] + acc_h

    pltpu.sync_copy(out_stage, out_hbm.at[wid])


@jax.jit
def _sc_loss_partials(dT, nT, h_flat, t_flat, d_tail, n_tail):
    mesh = plsc.VectorSubcoreMesh(core_axis_name="c", subcore_axis_name="s")
    return pl.kernel(
        _sc_body,
        out_type=jax.ShapeDtypeStruct((NW, 2, L), jnp.float32),
        mesh=mesh,
        compiler_params=pltpu.CompilerParams(needs_layout_passes=False),
        scratch_types=[
            pltpu.VMEM((N,), jnp.int32),
            pltpu.VMEM((2, KH, C), jnp.float32),
            pltpu.VMEM((2, KH, C), jnp.int32),
            pltpu.VMEM((2, C), jnp.float32),
            pltpu.VMEM((TAIL,), jnp.float32),
            pltpu.VMEM((TAIL * K,), jnp.float32),
            pltpu.VMEM((TAIL * K,), jnp.int32),
            pltpu.VMEM((2, L), jnp.float32),
            pltpu.SemaphoreType.DMA,
            pltpu.SemaphoreType.DMA,
        ],
    )(dT, nT, h_flat, t_flat, d_tail, n_tail)


def kernel(distances, hierarchy, neighbour_indices, truth_indices):
    assert distances.shape == (N, K)
    dT = distances.T
    nT = neighbour_indices.T
    h_flat = hierarchy.reshape(N)
    t_flat = truth_indices.reshape(N)
    d_tail = lax.slice(distances, (TAIL0, 0), (N, K)).reshape(TAIL * K)
    n_tail = lax.slice(neighbour_indices, (TAIL0, 0), (N, K)).reshape(TAIL * K)
    parts = _sc_loss_partials(dT, nT, h_flat, t_flat, d_tail, n_tail)
    pot_sum = jnp.sum(parts[:, 0, :])
    h_sum = jnp.sum(parts[:, 1, :])
    lossval = (1.0 - h_sum / N) + pot_sum / (N * K)
    return (distances, lossval)


# speedup vs baseline: 679.3077x; 1.0515x over previous
import jax
import jax.numpy as jnp
from jax import lax
from jax.experimental import pallas as pl
from jax.experimental.pallas import tpu as pltpu
from jax.experimental.pallas import tpu_sc as plsc

N = 100000
K = 64
NC, NS, L = 2, 16, 16
NW = NC * NS
C = 128
KH = K // 2
NBLK = N // C
TAIL0 = NBLK * C
TAIL = N - TAIL0
QMAX = (NBLK - 1) // NW + 1


def _sigmoid_h(x):
    return (1.0 / (1.0 + jnp.exp(-x)) + 1.0) * 0.5


def _sc_body(dT_hbm, nT_hbm, h_hbm, t_hbm, dtl_hbm, ntl_hbm, htl_hbm,
             ttl_hbm, out_hbm, dout_hbm,
             table_v, d_buf, n_buf, h_buf, dtl_v, ntl_v, htl_v, out_stage,
             sem_in0, sem_in1, sem_out0, sem_out1):
    wid = lax.axis_index("s") * NC + lax.axis_index("c")
    pltpu.sync_copy(t_hbm.at[0, pl.ds(0, TAIL0)], table_v.at[pl.ds(0, TAIL0)])
    pltpu.sync_copy(ttl_hbm, table_v.at[pl.ds(TAIL0, TAIL)])

    zero = jnp.zeros((L,), jnp.float32)
    nb = (NBLK - 1 - wid) // NW + 1
    sem_in = (sem_in0, sem_in1)
    sem_out = (sem_out0, sem_out1)

    def in_copies(q, half, buf):
        c0 = (wid + q * NW) * C
        return (
            (dT_hbm.at[pl.ds(half * KH, KH), pl.ds(c0, C)], d_buf.at[buf],
             sem_in[buf]),
            (nT_hbm.at[pl.ds(half * KH, KH), pl.ds(c0, C)], n_buf.at[buf],
             sem_in[buf]),
            (h_hbm.at[0, pl.ds(c0, C)], h_buf.at[buf], sem_in[buf]),
        )

    def start_in(q, half, buf):
        for src, dst, sem in in_copies(q, half, buf):
            pltpu.async_copy(src, dst, sem)

    def wait_in(q, half, buf):
        for src, dst, sem in in_copies(q, half, buf):
            pltpu.make_async_copy(src, dst, sem).wait()

    def out_copy(q, half, buf):
        c0 = (wid + q * NW) * C
        return (d_buf.at[buf],
                dout_hbm.at[pl.ds(half * KH, KH), pl.ds(c0, C)],
                sem_out[buf])

    def start_out(q, half, buf):
        src, dst, sem = out_copy(q, half, buf)
        pltpu.async_copy(src, dst, sem)

    def wait_out(q, half, buf):
        src, dst, sem = out_copy(q, half, buf)
        pltpu.make_async_copy(src, dst, sem).wait()

    def compute(q, buf, carry):
        c0 = (wid + q * NW) * C

        def g_body(g, carry):
            acc_pot, acc_pen = carry
            ftv = table_v[pl.ds(c0 + g * L, L)]
            hw = _sigmoid_h(h_buf[buf, pl.ds(g * L, L)])
            accs = [zero, zero, zero, zero]
            for k in range(KH):
                idx = n_buf[buf, k, pl.ds(g * L, L)]
                gt = plsc.load_gather(table_v, [idx])
                dv = d_buf[buf, k, pl.ds(g * L, L)]
                e = jnp.exp(dv * -3.0)
                accs[k & 3] = accs[k & 3] + jnp.where(ftv == gt,
                                                      0.5 - 0.5 * e, 0.5 * e)
            acc16 = (accs[0] + accs[1]) + (accs[2] + accs[3])
            return acc_pot + (hw * hw) * acc16, acc_pen + 0.5 * hw
        return lax.fori_loop(0, C // L, g_body, carry)

    start_in(0, 0, 0)

    def q_body(q, carry):
        @pl.when(jnp.logical_and(q >= 1, q < nb))
        def _w1():
            wait_out(q - 1, 1, 1)

        @pl.when(q < nb)
        def _p0():
            start_in(q, 1, 1)
            wait_in(q, 0, 0)
            start_out(q, 0, 0)
        carry0 = lax.cond(q < nb, lambda c: compute(q, 0, c),
                          lambda c: c, carry)
        @pl.when(q + 1 < nb)
        def _p1a():
            wait_out(q, 0, 0)
            start_in(q + 1, 0, 0)

        @pl.when(q < nb)
        def _p1b():
            wait_in(q, 1, 1)
            start_out(q, 1, 1)
        return lax.cond(q < nb, lambda c: compute(q, 1, c),
                        lambda c: c, carry0)

    acc_pot, acc_pen = lax.fori_loop(0, QMAX, q_body, (zero, zero))
    wait_out(nb - 1, 0, 0)
    wait_out(nb - 1, 1, 1)
    out_stage[0, :] = acc_pot
    out_stage[1, :] = acc_pen

    @pl.when(wid == NW - 1)
    def _tail():
        pltpu.sync_copy(dtl_hbm, dtl_v)
        pltpu.sync_copy(ntl_hbm, ntl_v)
        pltpu.sync_copy(htl_hbm, htl_v)

        acc_p, acc_h = zero, zero
        for g in range(TAIL // L):
            ftv = table_v[pl.ds(TAIL0 + g * L, L)]
            hw = _sigmoid_h(htl_v[pl.ds(g * L, L)])
            accs = [zero, zero, zero, zero]
            for k in range(K):
                idx = ntl_v[pl.ds(k * TAIL + g * L, L)]
                gt = plsc.load_gather(table_v, [idx])
                dv = dtl_v[pl.ds(k * TAIL + g * L, L)]
                e = jnp.exp(dv * -3.0)
                accs[k & 3] = accs[k & 3] + jnp.where(ftv == gt,
                                                      0.5 - 0.5 * e, 0.5 * e)
            acc16 = (accs[0] + accs[1]) + (accs[2] + accs[3])
            acc_p = acc_p + (hw * hw) * acc16
            acc_h = acc_h + hw
        out_stage[0, :] = out_stage[0, :] + acc_p
        out_stage[1, :] = out_stage[1, :] + acc_h

    pltpu.sync_copy(out_stage, out_hbm.at[wid])


@jax.jit
def _sc_loss_partials(dT, nT, hT, tT, d_tailT, n_tailT, h_tail, t_tail):
    mesh = plsc.VectorSubcoreMesh(core_axis_name="c", subcore_axis_name="s")
    return pl.kernel(
        _sc_body,
        out_type=(jax.ShapeDtypeStruct((NW, 2, L), jnp.float32),
                  jax.ShapeDtypeStruct((K, N), jnp.float32)),
        mesh=mesh,
        compiler_params=pltpu.CompilerParams(needs_layout_passes=False),
        scratch_types=[
            pltpu.VMEM((N,), jnp.int32),
            pltpu.VMEM((2, KH, C), jnp.float32),
            pltpu.VMEM((2, KH, C), jnp.int32),
            pltpu.VMEM((2, C), jnp.float32),
            pltpu.VMEM((K * TAIL,), jnp.float32),
            pltpu.VMEM((K * TAIL,), jnp.int32),
            pltpu.VMEM((TAIL,), jnp.float32),
            pltpu.VMEM((2, L), jnp.float32),
            pltpu.SemaphoreType.DMA,
            pltpu.SemaphoreType.DMA,
            pltpu.SemaphoreType.DMA,
            pltpu.SemaphoreType.DMA,
        ],
    )(dT, nT, hT, tT, d_tailT, n_tailT, h_tail, t_tail)


def kernel(distances, hierarchy, neighbour_indices, truth_indices):
    assert distances.shape == (N, K)
    dT = distances.T
    nT = neighbour_indices.T
    d_tailT = lax.slice(dT, (0, TAIL0), (K, N))
    n_tailT = lax.slice(nT, (0, TAIL0), (K, N))
    h_tail = lax.slice(hierarchy, (TAIL0, 0), (N, 1)).reshape(TAIL)
    t_tail = lax.slice(truth_indices, (TAIL0, 0), (N, 1)).reshape(TAIL)
    parts, d_out = _sc_loss_partials(
        dT, nT, hierarchy.T, truth_indices.T,
        d_tailT.reshape(K * TAIL), n_tailT.reshape(K * TAIL), h_tail, t_tail)
    d_out = lax.dynamic_update_slice(d_out, d_tailT, (0, TAIL0))
    pot_sum = jnp.sum(parts[:, 0, :])
    h_sum = jnp.sum(parts[:, 1, :])
    lossval = (1.0 - h_sum / N) + pot_sum / (N * K)
    return (d_out.T, lossval)


# speedup vs baseline: 741.0640x; 1.0909x over previous
import jax
import jax.numpy as jnp
from jax import lax
from jax.experimental import pallas as pl
from jax.experimental.pallas import tpu as pltpu
from jax.experimental.pallas import tpu_sc as plsc

N = 100000
K = 64
NC, NS, L = 2, 16, 16
NW = NC * NS
C = 128
KH = K // 2
NBLK = N // C
TAIL0 = NBLK * C
TAIL = N - TAIL0
QMAX = (NBLK - 1) // NW + 1


def _sigmoid_h(x):
    return (1.0 / (1.0 + jnp.exp(-x)) + 1.0) * 0.5


def _sc_body(dT_hbm, nT_hbm, h_hbm, t_hbm, dtl_hbm, ntl_hbm, htl_hbm,
             ttl_hbm, out_hbm, dout_hbm,
             table_v, d_buf, n_buf, h_buf, htl_v, out_stage,
             sem_in0, sem_in1, sem_in2, sem_out0, sem_out1, sem_out2):
    wid = lax.axis_index("s") * NC + lax.axis_index("c")
    pltpu.sync_copy(t_hbm.at[0, pl.ds(0, TAIL0)], table_v.at[pl.ds(0, TAIL0)])
    pltpu.sync_copy(ttl_hbm, table_v.at[pl.ds(TAIL0, TAIL)])

    zero = jnp.zeros((L,), jnp.float32)
    nb = (NBLK - 1 - wid) // NW + 1
    S = 2 * nb
    sem_in = (sem_in0, sem_in1, sem_in2)
    sem_out = (sem_out0, sem_out1, sem_out2)

    def step_slices(s):
        half = lax.rem(s, 2)
        c0 = (wid + (s // 2) * NW) * C
        return half, c0

    def in_copies(s, buf):
        half, c0 = step_slices(s)
        return (
            (dT_hbm.at[pl.ds(half * KH, KH), pl.ds(c0, C)], d_buf.at[buf],
             sem_in[buf]),
            (nT_hbm.at[pl.ds(half * KH, KH), pl.ds(c0, C)], n_buf.at[buf],
             sem_in[buf]),
            (h_hbm.at[0, pl.ds(c0, C)], h_buf.at[buf], sem_in[buf]),
        )

    def start_in(s, buf):
        for src, dst, sem in in_copies(s, buf):
            pltpu.async_copy(src, dst, sem)

    def wait_in(s, buf):
        for src, dst, sem in in_copies(s, buf):
            pltpu.make_async_copy(src, dst, sem).wait()

    def out_copy(s, buf):
        half, c0 = step_slices(s)
        return (d_buf.at[buf],
                dout_hbm.at[pl.ds(half * KH, KH), pl.ds(c0, C)],
                sem_out[buf])

    def start_out(s, buf):
        src, dst, sem = out_copy(s, buf)
        pltpu.async_copy(src, dst, sem)

    def wait_out(s, buf):
        src, dst, sem = out_copy(s, buf)
        pltpu.make_async_copy(src, dst, sem).wait()

    def compute(s, buf, carry):
        _, c0 = step_slices(s)

        def g_body(g, carry):
            acc_pot, acc_pen = carry
            ftv = table_v[pl.ds(c0 + g * L, L)]
            hw = _sigmoid_h(h_buf[buf, pl.ds(g * L, L)])
            accs = [zero, zero, zero, zero]
            for k in range(KH):
                idx = n_buf[buf, k, pl.ds(g * L, L)]
                gt = plsc.load_gather(table_v, [idx])
                dv = d_buf[buf, k, pl.ds(g * L, L)]
                e = jnp.exp(dv * -3.0)
                accs[k & 3] = accs[k & 3] + jnp.where(ftv == gt,
                                                      0.5 - 0.5 * e, 0.5 * e)
            acc16 = (accs[0] + accs[1]) + (accs[2] + accs[3])
            return acc_pot + (hw * hw) * acc16, acc_pen + 0.5 * hw
        return lax.fori_loop(0, C // L, g_body, carry)

    start_in(0, 0)
    start_in(1, 1)
    SMAX = 2 * QMAX

    def u_body(u, carry):
        for j in range(3):
            s = 3 * u + j
            buf = j

            @pl.when(jnp.logical_and(s >= 1, s - 1 < S))
            def _w():
                wait_out(s - 1, (buf + 2) % 3)

            @pl.when(s + 2 < S)
            def _pre():
                start_in(s + 2, (buf + 2) % 3)

            @pl.when(s < S)
            def _win():
                wait_in(s, buf)
                start_out(s, buf)
            carry = lax.cond(s < S, lambda c, s=s, buf=buf:
                             compute(s, buf, c), lambda c: c, carry)
        return carry

    acc_pot, acc_pen = lax.fori_loop(0, (SMAX + 3) // 3 + 1, u_body,
                                     (zero, zero))
    out_stage[0, :] = acc_pot
    out_stage[1, :] = acc_pen

    @pl.when(wid == NW - 1)
    def _tail():
        pltpu.sync_copy(dtl_hbm, d_buf.at[0, pl.ds(0, K * TAIL // C), :])
        pltpu.sync_copy(ntl_hbm, n_buf.at[0, pl.ds(0, K * TAIL // C), :])
        pltpu.sync_copy(htl_hbm, htl_v)

        acc_p, acc_h = zero, zero
        for g in range(TAIL // L):
            ftv = table_v[pl.ds(TAIL0 + g * L, L)]
            hw = _sigmoid_h(htl_v[pl.ds(g * L, L)])
            accs = [zero, zero, zero, zero]
            for k in range(K):
                p = k * TAIL + g * L
                idx = n_buf[0, p // C, pl.ds(p % C, L)]
                gt = plsc.load_gather(table_v, [idx])
                dv = d_buf[0, p // C, pl.ds(p % C, L)]
                e = jnp.exp(dv * -3.0)
                accs[k & 3] = accs[k & 3] + jnp.where(ftv == gt,
                                                      0.5 - 0.5 * e, 0.5 * e)
            acc16 = (accs[0] + accs[1]) + (accs[2] + accs[3])
            acc_p = acc_p + (hw * hw) * acc16
            acc_h = acc_h + hw
        out_stage[0, :] = out_stage[0, :] + acc_p
        out_stage[1, :] = out_stage[1, :] + acc_h

    pltpu.sync_copy(out_stage, out_hbm.at[wid])


@jax.jit
def _sc_loss_partials(dT, nT, hT, tT, d_tailT, n_tailT, h_tail, t_tail):
    mesh = plsc.VectorSubcoreMesh(core_axis_name="c", subcore_axis_name="s")
    return pl.kernel(
        _sc_body,
        out_type=(jax.ShapeDtypeStruct((NW, 2, L), jnp.float32),
                  jax.ShapeDtypeStruct((K, N), jnp.float32)),
        mesh=mesh,
        compiler_params=pltpu.CompilerParams(needs_layout_passes=False),
        scratch_types=[
            pltpu.VMEM((N,), jnp.int32),
            pltpu.VMEM((3, KH, C), jnp.float32),
            pltpu.VMEM((3, KH, C), jnp.int32),
            pltpu.VMEM((3, C), jnp.float32),
            pltpu.VMEM((TAIL,), jnp.float32),
            pltpu.VMEM((2, L), jnp.float32),
            pltpu.SemaphoreType.DMA,
            pltpu.SemaphoreType.DMA,
            pltpu.SemaphoreType.DMA,
            pltpu.SemaphoreType.DMA,
            pltpu.SemaphoreType.DMA,
            pltpu.SemaphoreType.DMA,
        ],
    )(dT, nT, hT, tT, d_tailT, n_tailT, h_tail, t_tail)


def kernel(distances, hierarchy, neighbour_indices, truth_indices):
    assert distances.shape == (N, K)
    dT = distances.T
    nT = neighbour_indices.T
    d_tailT = lax.slice(dT, (0, TAIL0), (K, N))
    n_tailT = lax.slice(nT, (0, TAIL0), (K, N))
    h_tail = lax.slice(hierarchy, (TAIL0, 0), (N, 1)).reshape(TAIL)
    t_tail = lax.slice(truth_indices, (TAIL0, 0), (N, 1)).reshape(TAIL)
    parts, d_out = _sc_loss_partials(
        dT, nT, hierarchy.T, truth_indices.T,
        d_tailT.reshape(K * TAIL // C, C), n_tailT.reshape(K * TAIL // C, C),
        h_tail, t_tail)
    d_out = lax.dynamic_update_slice(d_out, d_tailT, (0, TAIL0))
    pot_sum = jnp.sum(parts[:, 0, :])
    h_sum = jnp.sum(parts[:, 1, :])
    lossval = (1.0 - h_sum / N) + pot_sum / (N * K)
    return (d_out.T, lossval)


# speedup vs baseline: 745.4509x; 1.0059x over previous
import jax
import jax.numpy as jnp
from jax import lax
from jax.experimental import pallas as pl
from jax.experimental.pallas import tpu as pltpu
from jax.experimental.pallas import tpu_sc as plsc

N = 100000
K = 64
NC, NS, L = 2, 16, 16
NW = NC * NS
C = 128
KH = K // 2
NBLK = N // C
TAIL0 = NBLK * C
TAIL = N - TAIL0
QMAX = (NBLK - 1) // NW + 1


def _sigmoid_h(x):
    return (1.0 / (1.0 + jnp.exp(-x)) + 1.0) * 0.5


def _sc_body(dT_hbm, nT_hbm, h_hbm, t_hbm, pk_hbm, out_hbm, dout_hbm,
             table_v, d_buf, n_buf, h_buf, out_stage,
             sem_in0, sem_in1, sem_in2, sem_out0, sem_out1, sem_out2):
    wid = lax.axis_index("s") * NC + lax.axis_index("c")
    pltpu.sync_copy(t_hbm.at[0, pl.ds(0, TAIL0)], table_v.at[pl.ds(0, TAIL0)])
    pltpu.sync_copy(pk_hbm.at[pl.ds(2 * K * TAIL // C, 8), :],
                    n_buf.at[0, pl.ds(0, 8), :])
    for g in range(TAIL // L):
        table_v[pl.ds(TAIL0 + g * L, L)] = n_buf[0, 0,
                                                 pl.ds(TAIL + g * L, L)]

    zero = jnp.zeros((L,), jnp.float32)
    nb = (NBLK - 1 - wid) // NW + 1
    S = 2 * nb
    sem_in = (sem_in0, sem_in1, sem_in2)
    sem_out = (sem_out0, sem_out1, sem_out2)

    def step_slices(s):
        half = lax.rem(s, 2)
        c0 = (wid + (s // 2) * NW) * C
        return half, c0

    def in_copies(s, buf):
        half, c0 = step_slices(s)
        return (
            (dT_hbm.at[pl.ds(half * KH, KH), pl.ds(c0, C)], d_buf.at[buf],
             sem_in[buf]),
            (nT_hbm.at[pl.ds(half * KH, KH), pl.ds(c0, C)], n_buf.at[buf],
             sem_in[buf]),
            (h_hbm.at[0, pl.ds(c0, C)], h_buf.at[buf], sem_in[buf]),
        )

    def start_in(s, buf):
        for src, dst, sem in in_copies(s, buf):
            pltpu.async_copy(src, dst, sem)

    def wait_in(s, buf):
        for src, dst, sem in in_copies(s, buf):
            pltpu.make_async_copy(src, dst, sem).wait()

    def out_copy(s, buf):
        half, c0 = step_slices(s)
        return (d_buf.at[buf],
                dout_hbm.at[pl.ds(half * KH, KH), pl.ds(c0, C)],
                sem_out[buf])

    def start_out(s, buf):
        src, dst, sem = out_copy(s, buf)
        pltpu.async_copy(src, dst, sem)

    def wait_out(s, buf):
        src, dst, sem = out_copy(s, buf)
        pltpu.make_async_copy(src, dst, sem).wait()

    def compute(s, buf, carry):
        _, c0 = step_slices(s)

        def g_body(g, carry):
            acc_pot, acc_pen = carry
            ftv = table_v[pl.ds(c0 + g * L, L)]
            hw = _sigmoid_h(h_buf[buf, pl.ds(g * L, L)])
            accs = [zero, zero, zero, zero]
            for k in range(KH):
                idx = n_buf[buf, k, pl.ds(g * L, L)]
                gt = plsc.load_gather(table_v, [idx])
                dv = d_buf[buf, k, pl.ds(g * L, L)]
                e = jnp.exp(dv * -3.0)
                accs[k & 3] = accs[k & 3] + jnp.where(ftv == gt,
                                                      0.5 - 0.5 * e, 0.5 * e)
            acc16 = (accs[0] + accs[1]) + (accs[2] + accs[3])
            return acc_pot + (hw * hw) * acc16, acc_pen + 0.5 * hw
        return lax.fori_loop(0, C // L, g_body, carry)

    start_in(0, 0)
    start_in(1, 1)
    SMAX = 2 * QMAX

    def u_body(u, carry):
        for j in range(3):
            s = 3 * u + j
            buf = j

            @pl.when(jnp.logical_and(s >= 1, s - 1 < S))
            def _w():
                wait_out(s - 1, (buf + 2) % 3)

            @pl.when(s + 2 < S)
            def _pre():
                start_in(s + 2, (buf + 2) % 3)

            @pl.when(s < S)
            def _win():
                wait_in(s, buf)
                start_out(s, buf)
            carry = lax.cond(s < S, lambda c, s=s, buf=buf:
                             compute(s, buf, c), lambda c: c, carry)
        return carry

    acc_pot, acc_pen = lax.fori_loop(0, (SMAX + 3) // 3 + 1, u_body,
                                     (zero, zero))
    out_stage[0, :] = acc_pot
    out_stage[1, :] = acc_pen

    @pl.when(wid == NW - 1)
    def _tail():
        pltpu.sync_copy(pk_hbm.at[pl.ds(0, 2 * K * TAIL // C), :],
                        n_buf.at[0, :, :])
        pltpu.sync_copy(pk_hbm.at[pl.ds(2 * K * TAIL // C, 8), :],
                        n_buf.at[1, pl.ds(0, 8), :])

        acc_p, acc_h = zero, zero
        for g in range(TAIL // L):
            ftv = table_v[pl.ds(TAIL0 + g * L, L)]
            hw = _sigmoid_h(plsc.bitcast(n_buf[1, 0, pl.ds(g * L, L)],
                                         jnp.float32))
            accs = [zero, zero, zero, zero]
            for k in range(K):
                p = k * TAIL + g * L
                idx = n_buf[0, K * TAIL // C + p // C, pl.ds(p % C, L)]
                gt = plsc.load_gather(table_v, [idx])
                dv = plsc.bitcast(n_buf[0, p // C, pl.ds(p % C, L)],
                                  jnp.float32)
                e = jnp.exp(dv * -3.0)
                accs[k & 3] = accs[k & 3] + jnp.where(ftv == gt,
                                                      0.5 - 0.5 * e, 0.5 * e)
            acc16 = (accs[0] + accs[1]) + (accs[2] + accs[3])
            acc_p = acc_p + (hw * hw) * acc16
            acc_h = acc_h + hw
        out_stage[0, :] = out_stage[0, :] + acc_p
        out_stage[1, :] = out_stage[1, :] + acc_h

    pltpu.sync_copy(out_stage, out_hbm.at[wid])


@jax.jit
def _sc_loss_partials(dT, nT, hT, tT, packed_tail):
    mesh = plsc.VectorSubcoreMesh(core_axis_name="c", subcore_axis_name="s")
    return pl.kernel(
        _sc_body,
        out_type=(jax.ShapeDtypeStruct((NW, 2, L), jnp.float32),
                  jax.ShapeDtypeStruct((K, N), jnp.float32)),
        mesh=mesh,
        compiler_params=pltpu.CompilerParams(needs_layout_passes=False),
        scratch_types=[
            pltpu.VMEM((N,), jnp.int32),
            pltpu.VMEM((3, KH, C), jnp.float32),
            pltpu.VMEM((3, KH, C), jnp.int32),
            pltpu.VMEM((3, C), jnp.float32),
            pltpu.VMEM((2, L), jnp.float32),
            pltpu.SemaphoreType.DMA,
            pltpu.SemaphoreType.DMA,
            pltpu.SemaphoreType.DMA,
            pltpu.SemaphoreType.DMA,
            pltpu.SemaphoreType.DMA,
            pltpu.SemaphoreType.DMA,
        ],
    )(dT, nT, hT, tT, packed_tail)


def kernel(distances, hierarchy, neighbour_indices, truth_indices):
    assert distances.shape == (N, K)
    dT = distances.T
    nT = neighbour_indices.T
    d_tailT = lax.slice(dT, (0, TAIL0), (K, N))
    n_tailT = lax.slice(nT, (0, TAIL0), (K, N))
    h_tail = lax.slice(hierarchy, (TAIL0, 0), (N, 1)).reshape(TAIL)
    t_tail = lax.slice(truth_indices, (TAIL0, 0), (N, 1)).reshape(TAIL)
    packed_tail = jnp.concatenate([
        lax.bitcast_convert_type(d_tailT, jnp.int32).reshape(
            K * TAIL // C, C),
        n_tailT.reshape(K * TAIL // C, C),
        jnp.concatenate([lax.bitcast_convert_type(h_tail, jnp.int32),
                         t_tail,
                         jnp.zeros((C - 2 * TAIL,), jnp.int32)])[None, :],
        jnp.zeros((7, C), jnp.int32),
    ], axis=0)
    parts, d_out = _sc_loss_partials(
        dT, nT, hierarchy.T, truth_indices.T, packed_tail)
    d_out = lax.dynamic_update_slice(d_out, d_tailT, (0, TAIL0))
    pot_sum = jnp.sum(parts[:, 0, :])
    h_sum = jnp.sum(parts[:, 1, :])
    lossval = (1.0 - h_sum / N) + pot_sum / (N * K)
    return (d_out.T, lossval)


# speedup vs baseline: 762.1182x; 1.0224x over previous
import jax
import jax.numpy as jnp
from jax import lax
from jax.experimental import pallas as pl
from jax.experimental.pallas import tpu as pltpu
from jax.experimental.pallas import tpu_sc as plsc

N = 100000
K = 64
NC, NS, L = 2, 16, 16
NW = NC * NS
C = 128
KH = K // 2
NBLK = N // C
TAIL0 = NBLK * C
TAIL = N - TAIL0
QMAX = (NBLK - 1) // NW + 1


def _sigmoid_h(x):
    return (1.0 / (1.0 + jnp.exp(-x)) + 1.0) * 0.5


def _sc_body(dT_hbm, nT_hbm, h_hbm, t_hbm, pk_hbm, out_hbm, dout_hbm,
             table_v, d_buf, n_buf, h_buf, out_stage,
             sem_in0, sem_in1, sem_in2, sem_out0, sem_out1, sem_out2):
    wid = lax.axis_index("s") * NC + lax.axis_index("c")
    pltpu.sync_copy(t_hbm.at[0, pl.ds(0, TAIL0)], table_v.at[pl.ds(0, TAIL0)])
    pltpu.sync_copy(pk_hbm.at[pl.ds(2 * K * TAIL // C, 8), :],
                    n_buf.at[0, pl.ds(0, 8), :])
    for g in range(TAIL // L):
        table_v[pl.ds(TAIL0 + g * L, L)] = n_buf[0, 0,
                                                 pl.ds(TAIL + g * L, L)]

    zero = jnp.zeros((L,), jnp.float32)
    nb = (NBLK - 1 - wid) // NW + 1
    S = 2 * nb
    sem_in = (sem_in0, sem_in1, sem_in2)
    sem_out = (sem_out0, sem_out1, sem_out2)

    def step_slices(s):
        half = lax.rem(s, 2)
        c0 = (wid + (s // 2) * NW) * C
        return half, c0

    def in_copies(s, buf):
        half, c0 = step_slices(s)
        return (
            (dT_hbm.at[pl.ds(half * KH, KH), pl.ds(c0, C)], d_buf.at[buf],
             sem_in[buf]),
            (nT_hbm.at[pl.ds(half * KH, KH), pl.ds(c0, C)], n_buf.at[buf],
             sem_in[buf]),
            (h_hbm.at[0, pl.ds(c0, C)], h_buf.at[buf], sem_in[buf]),
        )

    def start_in(s, buf):
        for src, dst, sem in in_copies(s, buf):
            pltpu.async_copy(src, dst, sem)

    def wait_in(s, buf):
        for src, dst, sem in in_copies(s, buf):
            pltpu.make_async_copy(src, dst, sem).wait()

    def out_copy(s, buf):
        half, c0 = step_slices(s)
        return (d_buf.at[buf],
                dout_hbm.at[pl.ds(half * KH, KH), pl.ds(c0, C)],
                sem_out[buf])

    def start_out(s, buf):
        src, dst, sem = out_copy(s, buf)
        pltpu.async_copy(src, dst, sem)

    def wait_out(s, buf):
        src, dst, sem = out_copy(s, buf)
        pltpu.make_async_copy(src, dst, sem).wait()

    def compute(s, buf, carry):
        _, c0 = step_slices(s)

        def g_body(g, carry):
            acc_pot, acc_pen = carry
            ftv = table_v[pl.ds(c0 + g * L, L)]
            hw = _sigmoid_h(h_buf[buf, pl.ds(g * L, L)])
            accs = [zero, zero, zero, zero]
            for k in range(KH):
                idx = n_buf[buf, k, pl.ds(g * L, L)]
                gt = plsc.load_gather(table_v, [idx])
                dv = d_buf[buf, k, pl.ds(g * L, L)]
                e = jnp.exp(dv * -3.0)
                accs[k & 3] = accs[k & 3] + jnp.where(ftv == gt, 1.0 - e, e)
            acc16 = (accs[0] + accs[1]) + (accs[2] + accs[3])
            return acc_pot + (0.5 * hw * hw) * acc16, acc_pen + 0.5 * hw
        return lax.fori_loop(0, C // L, g_body, carry)

    start_in(0, 0)
    start_in(1, 1)
    SMAX = 2 * QMAX

    def u_body(u, carry):
        for j in range(3):
            s = 3 * u + j
            buf = j

            @pl.when(jnp.logical_and(s >= 1, s - 1 < S))
            def _w():
                wait_out(s - 1, (buf + 2) % 3)

            @pl.when(s + 2 < S)
            def _pre():
                start_in(s + 2, (buf + 2) % 3)

            @pl.when(s < S)
            def _win():
                wait_in(s, buf)
                start_out(s, buf)
            carry = lax.cond(s < S, lambda c, s=s, buf=buf:
                             compute(s, buf, c), lambda c: c, carry)
        return carry

    acc_pot, acc_pen = lax.fori_loop(0, (SMAX + 3) // 3 + 1, u_body,
                                     (zero, zero))
    out_stage[0, :] = acc_pot
    out_stage[1, :] = acc_pen

    @pl.when(wid == NW - 1)
    def _tail():
        pltpu.sync_copy(pk_hbm.at[pl.ds(0, 2 * K * TAIL // C), :],
                        n_buf.at[0, :, :])
        pltpu.sync_copy(pk_hbm.at[pl.ds(2 * K * TAIL // C, 8), :],
                        n_buf.at[1, pl.ds(0, 8), :])

        acc_p, acc_h = zero, zero
        for g in range(TAIL // L):
            ftv = table_v[pl.ds(TAIL0 + g * L, L)]
            hw = _sigmoid_h(plsc.bitcast(n_buf[1, 0, pl.ds(g * L, L)],
                                         jnp.float32))
            accs = [zero, zero, zero, zero]
            for k in range(K):
                p = k * TAIL + g * L
                idx = n_buf[0, K * TAIL // C + p // C, pl.ds(p % C, L)]
                gt = plsc.load_gather(table_v, [idx])
                dv = plsc.bitcast(n_buf[0, p // C, pl.ds(p % C, L)],
                                  jnp.float32)
                e = jnp.exp(dv * -3.0)
                accs[k & 3] = accs[k & 3] + jnp.where(ftv == gt, 1.0 - e, e)
            acc16 = (accs[0] + accs[1]) + (accs[2] + accs[3])
            acc_p = acc_p + (0.5 * hw * hw) * acc16
            acc_h = acc_h + hw
        out_stage[0, :] = out_stage[0, :] + acc_p
        out_stage[1, :] = out_stage[1, :] + acc_h

    pltpu.sync_copy(out_stage, out_hbm.at[wid])


@jax.jit
def _sc_loss_partials(dT, nT, hT, tT, packed_tail):
    mesh = plsc.VectorSubcoreMesh(core_axis_name="c", subcore_axis_name="s")
    return pl.kernel(
        _sc_body,
        out_type=(jax.ShapeDtypeStruct((NW, 2, L), jnp.float32),
                  jax.ShapeDtypeStruct((K, N), jnp.float32)),
        mesh=mesh,
        compiler_params=pltpu.CompilerParams(needs_layout_passes=False),
        scratch_types=[
            pltpu.VMEM((N,), jnp.int32),
            pltpu.VMEM((3, KH, C), jnp.float32),
            pltpu.VMEM((3, KH, C), jnp.int32),
            pltpu.VMEM((3, C), jnp.float32),
            pltpu.VMEM((2, L), jnp.float32),
            pltpu.SemaphoreType.DMA,
            pltpu.SemaphoreType.DMA,
            pltpu.SemaphoreType.DMA,
            pltpu.SemaphoreType.DMA,
            pltpu.SemaphoreType.DMA,
            pltpu.SemaphoreType.DMA,
        ],
    )(dT, nT, hT, tT, packed_tail)


def kernel(distances, hierarchy, neighbour_indices, truth_indices):
    assert distances.shape == (N, K)
    dT = distances.T
    nT = neighbour_indices.T
    d_tailT = lax.slice(dT, (0, TAIL0), (K, N))
    n_tailT = lax.slice(nT, (0, TAIL0), (K, N))
    h_tail = lax.slice(hierarchy, (TAIL0, 0), (N, 1)).reshape(TAIL)
    t_tail = lax.slice(truth_indices, (TAIL0, 0), (N, 1)).reshape(TAIL)
    packed_tail = jnp.concatenate([
        lax.bitcast_convert_type(d_tailT, jnp.int32).reshape(
            K * TAIL // C, C),
        n_tailT.reshape(K * TAIL // C, C),
        jnp.concatenate([lax.bitcast_convert_type(h_tail, jnp.int32),
                         t_tail,
                         jnp.zeros((C - 2 * TAIL,), jnp.int32)])[None, :],
        jnp.zeros((7, C), jnp.int32),
    ], axis=0)
    parts, d_out = _sc_loss_partials(
        dT, nT, hierarchy.T, truth_indices.T, packed_tail)
    d_out = lax.dynamic_update_slice(d_out, d_tailT, (0, TAIL0))
    pot_sum = jnp.sum(parts[:, 0, :])
    h_sum = jnp.sum(parts[:, 1, :])
    lossval = (1.0 - h_sum / N) + pot_sum / (N * K)
    return (d_out.T, lossval)
